# Initial kernel scaffold; baseline (speedup 1.0000x reference)
import jax
import jax.numpy as jnp
from jax.experimental import pallas as pl


def kernel(x, pe_table):
    raise NotImplementedError("write your pallas kernel here")


# speedup vs baseline: 1.3095x; 1.3095x over previous
import jax
import jax.numpy as jnp
from jax.experimental import pallas as pl
from jax.experimental.pallas import tpu as pltpu

BLOCK_T = 1024


def _add_body(x_ref, pe_ref, o_ref):
    o_ref[...] = x_ref[...] + pe_ref[...]


def kernel(x, pe_table):
    B, T, D = x.shape
    grid = (B, T // BLOCK_T)
    return pl.pallas_call(
        _add_body,
        grid=grid,
        in_specs=[
            pl.BlockSpec((1, BLOCK_T, D), lambda b, t: (b, t, 0)),
            pl.BlockSpec((BLOCK_T, D), lambda b, t: (t, 0)),
        ],
        out_specs=pl.BlockSpec((1, BLOCK_T, D), lambda b, t: (b, t, 0)),
        out_shape=jax.ShapeDtypeStruct((B, T, D), x.dtype),
    )(x, pe_table)


# speedup vs baseline: 1.6684x; 1.2741x over previous
import jax
import jax.numpy as jnp
from jax.experimental import pallas as pl
from jax.experimental.pallas import tpu as pltpu

BLOCK_T = 1024


def _add_body(x_ref, pe_ref, o_ref):
    o_ref[...] = x_ref[...] + pe_ref[...]


def kernel(x, pe_table):
    B, T, D = x.shape
    grid = (T // BLOCK_T, B)
    return pl.pallas_call(
        _add_body,
        grid=grid,
        in_specs=[
            pl.BlockSpec((1, BLOCK_T, D), lambda t, b: (b, t, 0)),
            pl.BlockSpec((BLOCK_T, D), lambda t, b: (t, 0)),
        ],
        out_specs=pl.BlockSpec((1, BLOCK_T, D), lambda t, b: (b, t, 0)),
        out_shape=jax.ShapeDtypeStruct((B, T, D), x.dtype),
    )(x, pe_table)
